# baseline (speedup 1.0000x reference)
import functools

import jax
import jax.numpy as jnp
from jax import lax
from jax.experimental import pallas as pl
from jax.experimental.pallas import tpu as pltpu
from jax.experimental.pallas import tpu_sc as plsc

NUM_CORES = 2
NUM_SUBCORES = 16
LANES = 16
NUM_WORKERS = NUM_CORES * NUM_SUBCORES

ROWS_PER_CHUNK = 32
NBUF = 4


@functools.partial(jax.jit, static_argnums=(2, 3))
def _onehot_sc(zeros_flat, idx, batch, nb_digits):
    rows_per_worker = batch // NUM_WORKERS
    chunks = rows_per_worker // ROWS_PER_CHUNK
    chunk_elems = ROWS_PER_CHUNK * nb_digits

    mesh = plsc.VectorSubcoreMesh(core_axis_name="c", subcore_axis_name="s")

    def body(zeros_hbm, idx_hbm, out_hbm, idx_v, *bufs_sems):
        bufs, sems = bufs_sems[:NBUF], bufs_sems[NBUF:]
        wid = lax.axis_index("s") * NUM_CORES + lax.axis_index("c")
        row_base = wid * rows_per_worker
        elem_base = row_base * nb_digits

        pltpu.sync_copy(idx_hbm.at[pl.ds(row_base, rows_per_worker)], idx_v)
        init = [
            pltpu.async_copy(zeros_hbm.at[pl.ds(0, chunk_elems)], bufs[b],
                             sems[b])
            for b in range(NBUF)
        ]

        iota = lax.iota(jnp.int32, LANES)
        ones = jnp.full((LANES,), 1.0, jnp.float32)
        zeros = jnp.zeros((LANES,), jnp.float32)

        def flat_pos(c, j):
            col_v = idx_v[pl.ds(c * ROWS_PER_CHUNK + j * LANES, LANES)]
            return (iota + j * LANES) * nb_digits + col_v

        descs = [None] * chunks
        for c in range(chunks):
            b = c % NBUF
            if c < NBUF:
                init[b].wait()
            else:
                descs[c - NBUF].wait()
                for j in range(ROWS_PER_CHUNK // LANES):
                    plsc.store_scatter(bufs[b], [flat_pos(c - NBUF, j)], zeros)
            for j in range(ROWS_PER_CHUNK // LANES):
                plsc.store_scatter(bufs[b], [flat_pos(c, j)], ones)
            descs[c] = pltpu.async_copy(
                bufs[b],
                out_hbm.at[pl.ds(elem_base + c * chunk_elems, chunk_elems)],
                sems[b])
        for c in range(chunks - NBUF, chunks):
            descs[c].wait()

    f = pl.kernel(
        body,
        out_type=jax.ShapeDtypeStruct((batch * nb_digits,), jnp.float32),
        mesh=mesh,
        scratch_types=(
            [pltpu.VMEM((rows_per_worker,), jnp.int32)]
            + [pltpu.VMEM((chunk_elems,), jnp.float32) for _ in range(NBUF)]
            + [pltpu.SemaphoreType.DMA for _ in range(NBUF)]
        ),
        compiler_params=pltpu.CompilerParams(needs_layout_passes=False),
    )
    return f(zeros_flat, idx)


def kernel(onehot_buf, long_tensor, nb_digits):
    del nb_digits
    batch, digits = onehot_buf.shape
    idx = long_tensor.reshape(-1).astype(jnp.int32)
    flat = _onehot_sc(onehot_buf.reshape(-1), idx, batch, digits)
    return flat.reshape(batch, digits)


# speedup vs baseline: 1.0043x; 1.0043x over previous
import functools

import jax
import jax.numpy as jnp
from jax import lax
from jax.experimental import pallas as pl
from jax.experimental.pallas import tpu as pltpu
from jax.experimental.pallas import tpu_sc as plsc

NUM_CORES = 2
NUM_SUBCORES = 16
LANES = 16
NUM_WORKERS = NUM_CORES * NUM_SUBCORES

ROWS_PER_CHUNK = 32
NBUF = 4


@functools.partial(jax.jit, static_argnums=(2, 3))
def _onehot_sc(zeros_buf, idx, batch, nb_digits):
    rows_per_worker = batch // NUM_WORKERS
    chunks = rows_per_worker // ROWS_PER_CHUNK

    mesh = plsc.VectorSubcoreMesh(core_axis_name="c", subcore_axis_name="s")

    def body(zeros_hbm, idx_hbm, out_hbm, idx_v, *bufs_sems):
        bufs, sems = bufs_sems[:NBUF], bufs_sems[NBUF:]
        wid = lax.axis_index("s") * NUM_CORES + lax.axis_index("c")
        row_base = wid * rows_per_worker

        pltpu.sync_copy(idx_hbm.at[pl.ds(row_base, rows_per_worker)], idx_v)
        init = [
            pltpu.async_copy(zeros_hbm.at[pl.ds(0, ROWS_PER_CHUNK)], bufs[b],
                             sems[b])
            for b in range(NBUF)
        ]

        iota = lax.iota(jnp.int32, LANES)
        ones = jnp.full((LANES,), 1.0, jnp.float32)
        zeros = jnp.zeros((LANES,), jnp.float32)

        def pos(c, j):
            col_v = idx_v[pl.ds(c * ROWS_PER_CHUNK + j * LANES, LANES)]
            return [iota + j * LANES, col_v]

        descs = [None] * chunks
        for c in range(chunks):
            b = c % NBUF
            if c < NBUF:
                init[b].wait()
            else:
                descs[c - NBUF].wait()
                for j in range(ROWS_PER_CHUNK // LANES):
                    plsc.store_scatter(bufs[b], pos(c - NBUF, j), zeros)
            for j in range(ROWS_PER_CHUNK // LANES):
                plsc.store_scatter(bufs[b], pos(c, j), ones)
            descs[c] = pltpu.async_copy(
                bufs[b],
                out_hbm.at[pl.ds(row_base + c * ROWS_PER_CHUNK,
                                 ROWS_PER_CHUNK)],
                sems[b])
        for c in range(chunks - NBUF, chunks):
            descs[c].wait()

    f = pl.kernel(
        body,
        out_type=jax.ShapeDtypeStruct((batch, nb_digits), jnp.float32),
        mesh=mesh,
        scratch_types=(
            [pltpu.VMEM((rows_per_worker,), jnp.int32)]
            + [pltpu.VMEM((ROWS_PER_CHUNK, nb_digits), jnp.float32)
               for _ in range(NBUF)]
            + [pltpu.SemaphoreType.DMA for _ in range(NBUF)]
        ),
        compiler_params=pltpu.CompilerParams(
            needs_layout_passes=False,
            use_tc_tiling_on_sc=False,
        ),
    )
    return f(zeros_buf, idx)


def kernel(onehot_buf, long_tensor, nb_digits):
    del nb_digits
    batch, digits = onehot_buf.shape
    idx = long_tensor.reshape(-1).astype(jnp.int32)
    return _onehot_sc(onehot_buf, idx, batch, digits)


# speedup vs baseline: 4.4027x; 4.3837x over previous
import functools

import jax
import jax.numpy as jnp
from jax import lax
from jax.experimental import pallas as pl
from jax.experimental.pallas import tpu as pltpu
from jax.experimental.pallas import tpu_sc as plsc

NUM_CORES = 2
NUM_SUBCORES = 16
LANES = 16
NUM_WORKERS = NUM_CORES * NUM_SUBCORES

SUBLANE = 8
LANE128 = 128

T1C = 5
NBUF = 4
UNROLL = 4


@functools.partial(jax.jit, static_argnums=(2, 3))
def _onehot_sc(idx, zsrc, batch, nb_digits):
    rows_per_worker = batch // NUM_WORKERS
    t0n = batch // LANE128
    t0_per_worker = t0n // NUM_WORKERS
    num_t1 = nb_digits // SUBLANE
    chunks = num_t1 // T1C
    digits_per_chunk = T1C * SUBLANE
    groups = rows_per_worker // LANES

    mesh = plsc.VectorSubcoreMesh(core_axis_name="c", subcore_axis_name="s")

    def body(idx_hbm, zsrc_hbm, out_hbm, idx_v, *bufs_sems):
        bufs, sems = bufs_sems[:NBUF], bufs_sems[NBUF:]
        wid = lax.axis_index("s") * NUM_CORES + lax.axis_index("c")
        row_base = wid * rows_per_worker
        t0_base = wid * t0_per_worker

        pltpu.sync_copy(idx_hbm.at[pl.ds(row_base, rows_per_worker)], idx_v)
        init = [pltpu.async_copy(zsrc_hbm, bufs[b], sems[b])
                for b in range(NBUF)]

        iota = lax.iota(jnp.int32, LANES)
        ones = jnp.full((LANES,), 1.0, jnp.float32)
        zeros = jnp.zeros((LANES,), jnp.float32)

        def scan_chunk(k, buf, value):
            lo = k * digits_per_chunk

            def it(g0, _):
                for u in range(UNROLL):
                    g = g0 * UNROLL + u
                    col_v = idx_v[pl.ds(g * LANES, LANES)]
                    m = jnp.logical_and(col_v >= lo,
                                        col_v < lo + digits_per_chunk)
                    t1_v = lax.shift_right_logical(col_v, 3) - k * T1C
                    a_v = lax.bitwise_and(col_v, 7)
                    t0_v = jnp.full((LANES,), lax.shift_right_logical(g, 3),
                                    jnp.int32)
                    b_v = lax.shift_left(lax.bitwise_and(g, 7), 4) + iota
                    plsc.store_scatter(buf, [t1_v, t0_v, a_v, b_v], value,
                                       mask=m)
                return 0

            lax.fori_loop(0, groups // UNROLL, it, 0, unroll=False)

        descs = [None] * chunks
        for c in range(chunks):
            b = c % NBUF
            if c < NBUF:
                init[b].wait()
            else:
                descs[c - NBUF].wait()
                scan_chunk(c - NBUF, bufs[b], zeros)
            scan_chunk(c, bufs[b], ones)
            descs[c] = pltpu.async_copy(
                bufs[b],
                out_hbm.at[pl.ds(c * T1C, T1C), pl.ds(t0_base, t0_per_worker)],
                sems[b])
        for c in range(chunks - NBUF, chunks):
            descs[c].wait()

    f = pl.kernel(
        body,
        out_type=jax.ShapeDtypeStruct((num_t1, t0n, SUBLANE, LANE128),
                                      jnp.float32),
        mesh=mesh,
        scratch_types=(
            [pltpu.VMEM((rows_per_worker,), jnp.int32)]
            + [pltpu.VMEM((T1C, t0_per_worker, SUBLANE, LANE128), jnp.float32)
               for _ in range(NBUF)]
            + [pltpu.SemaphoreType.DMA for _ in range(NBUF)]
        ),
        compiler_params=pltpu.CompilerParams(
            needs_layout_passes=False,
            use_tc_tiling_on_sc=False,
        ),
    )
    t = f(idx, zsrc)
    return t.transpose(1, 3, 0, 2).reshape(batch, nb_digits)


def kernel(onehot_buf, long_tensor, nb_digits):
    del nb_digits
    batch, digits = onehot_buf.shape
    idx = long_tensor.reshape(-1).astype(jnp.int32)
    zsrc = jnp.zeros((T1C, (batch // LANE128) // NUM_WORKERS, SUBLANE,
                      LANE128), jnp.float32)
    return _onehot_sc(idx, zsrc, batch, digits)
